# baseline (speedup 1.0000x reference)
import jax
import jax.numpy as jnp
from jax.experimental import pallas as pl


def _pos_embed_kernel(emb_ref, out_ref):
    out_ref[0] = emb_ref[...].T


def kernel(x, mask, row_embed):
    B = x.shape[0]
    F = x.shape[1]
    L = x.shape[-1]
    emb = row_embed[:L]
    return pl.pallas_call(
        _pos_embed_kernel,
        grid=(B,),
        in_specs=[pl.BlockSpec((L, F), lambda b: (0, 0))],
        out_specs=pl.BlockSpec((1, F, L), lambda b: (b, 0, 0)),
        out_shape=jax.ShapeDtypeStruct((B, F, L), jnp.float32),
    )(emb)


# speedup vs baseline: 1.4863x; 1.4863x over previous
import jax
import jax.numpy as jnp
from jax.experimental import pallas as pl


def _pos_embed_kernel(emb_ref, out_ref):
    t = emb_ref[...].T
    out_ref[...] = jnp.broadcast_to(t[None], out_ref.shape)


def kernel(x, mask, row_embed):
    B = x.shape[0]
    F = x.shape[1]
    L = x.shape[-1]
    LT = 256
    return pl.pallas_call(
        _pos_embed_kernel,
        grid=(L // LT,),
        in_specs=[pl.BlockSpec((LT, F), lambda l: (l, 0))],
        out_specs=pl.BlockSpec((B, F, LT), lambda l: (0, 0, l)),
        out_shape=jax.ShapeDtypeStruct((B, F, L), jnp.float32),
    )(row_embed)
